# Initial kernel scaffold; baseline (speedup 1.0000x reference)
import functools

import jax
import jax.numpy as jnp
from jax import lax
from jax.experimental import pallas as pl
from jax.experimental.pallas import tpu as pltpu
from jax.experimental.pallas import tpu_sc as plsc

DIM = 64

NC = 2
NS = 16
NW = NC * NS

GB = 128
NB_W = 104
NBATCH = NW * NB_W
NPAD = NBATCH * GB


def _fuse_tables_body(wc_ref, wch_ref, ws_ref, wl_ref, b_ref, out_ref):
    wl = wl_ref[...]
    t_c = jnp.dot(wc_ref[...], wl[0:64], preferred_element_type=jnp.float32)
    t_ch = jnp.dot(wch_ref[...], wl[64:128], preferred_element_type=jnp.float32)
    t_s = jnp.dot(ws_ref[...], wl[128:192], preferred_element_type=jnp.float32)
    out_ref[0:16, :] = t_c + b_ref[...]
    out_ref[16:272, :] = t_ch
    out_ref[272:528, :] = t_s


def _fuse_tables(W_colors, W_chars, W_specials, W_lin, b_lin):
    return pl.pallas_call(
        _fuse_tables_body,
        out_shape=jax.ShapeDtypeStruct((16 + 256 + 256, DIM), jnp.float32),
    )(W_colors, W_chars, W_specials, W_lin, b_lin.reshape(1, DIM))


def _sc_gather_body(idxc_h, idxch_h, idxs_h, table_h, out_h,
                    idxc_v, idxch_v, idxs_v, rows_v, sem):
    wid = lax.axis_index("s") * NC + lax.axis_index("c")
    b0 = wid * NB_W
    pltpu.sync_copy(idxc_h.at[pl.ds(b0, NB_W)], idxc_v)
    pltpu.sync_copy(idxch_h.at[pl.ds(b0, NB_W)], idxch_v)
    pltpu.sync_copy(idxs_h.at[pl.ds(b0, NB_W)], idxs_v)

    def step(j, carry):
        pltpu.async_copy(table_h.at[idxc_v.at[j]], rows_v, sem).wait()
        pltpu.async_copy(table_h.at[idxch_v.at[j]], rows_v, sem, add=True).wait()
        pltpu.async_copy(table_h.at[idxs_v.at[j]], rows_v, sem, add=True).wait()
        pltpu.sync_copy(rows_v, out_h.at[pl.ds((b0 + j) * GB, GB)])
        return carry

    lax.fori_loop(0, NB_W, step, 0)


_sc_gather = functools.partial(
    pl.kernel,
    _sc_gather_body,
    out_type=jax.ShapeDtypeStruct((NPAD, DIM), jnp.float32),
    mesh=plsc.VectorSubcoreMesh(core_axis_name="c", subcore_axis_name="s"),
    scratch_types=[
        pltpu.VMEM((NB_W, GB), jnp.int32),
        pltpu.VMEM((NB_W, GB), jnp.int32),
        pltpu.VMEM((NB_W, GB), jnp.int32),
        pltpu.VMEM((GB, DIM), jnp.float32),
        pltpu.SemaphoreType.DMA,
    ],
)()


def kernel(colors, chars, specials, W_colors, W_chars, W_specials, W_lin, b_lin):
    table = _fuse_tables(W_colors, W_chars, W_specials, W_lin, b_lin)

    n = colors.size
    pad = NPAD - n
    idxc = jnp.pad(colors.reshape(-1).astype(jnp.int32), (0, pad))
    idxch = jnp.pad(chars.reshape(-1).astype(jnp.int32) + 16, (0, pad))
    idxs = jnp.pad(specials.reshape(-1).astype(jnp.int32) + 272, (0, pad))

    out = _sc_gather(
        idxc.reshape(NBATCH, GB),
        idxch.reshape(NBATCH, GB),
        idxs.reshape(NBATCH, GB),
        table,
    )
    return out[:n].reshape(colors.shape + (DIM,))


# speedup vs baseline: 1.3962x; 1.3962x over previous
import functools

import jax
import jax.numpy as jnp
from jax import lax
from jax.experimental import pallas as pl
from jax.experimental.pallas import tpu as pltpu
from jax.experimental.pallas import tpu_sc as plsc

DIM = 64

NC = 2
NS = 16
NW = NC * NS

GB = 128
NB_W = 104
NBATCH = NW * NB_W
NPAD = NBATCH * GB


def _fuse_tables_body(wc_ref, wch_ref, ws_ref, wl_ref, b_ref, out_ref):
    wl = wl_ref[...]
    t_c = jnp.dot(wc_ref[...], wl[0:64], preferred_element_type=jnp.float32)
    t_ch = jnp.dot(wch_ref[...], wl[64:128], preferred_element_type=jnp.float32)
    t_s = jnp.dot(ws_ref[...], wl[128:192], preferred_element_type=jnp.float32)
    out_ref[0:16, :] = t_c + b_ref[...]
    out_ref[16:272, :] = t_ch
    out_ref[272:528, :] = t_s


def _fuse_tables(W_colors, W_chars, W_specials, W_lin, b_lin):
    return pl.pallas_call(
        _fuse_tables_body,
        out_shape=jax.ShapeDtypeStruct((16 + 256 + 256, DIM), jnp.float32),
    )(W_colors, W_chars, W_specials, W_lin, b_lin.reshape(1, DIM))


def _sc_gather_body(idxc_h, idxch_h, idxs_h, table_h, out_h,
                    idxc_v, idxch_v, idxs_v, rows_v, sem):
    wid = lax.axis_index("s") * NC + lax.axis_index("c")
    b0 = wid * NB_W
    pltpu.sync_copy(idxc_h.at[pl.ds(b0, NB_W)], idxc_v)
    pltpu.sync_copy(idxch_h.at[pl.ds(b0, NB_W)], idxch_v)
    pltpu.sync_copy(idxs_h.at[pl.ds(b0, NB_W)], idxs_v)

    def step(j, carry):
        pltpu.async_copy(table_h.at[idxc_v.at[j]], rows_v, sem).wait()
        pltpu.async_copy(table_h.at[idxch_v.at[j]], rows_v, sem, add=True).wait()
        pltpu.async_copy(table_h.at[idxs_v.at[j]], rows_v, sem, add=True).wait()
        pltpu.sync_copy(rows_v, out_h.at[pl.ds((b0 + j) * GB, GB)])
        return carry

    lax.fori_loop(0, NB_W, step, 0)


_sc_gather = functools.partial(
    pl.kernel,
    _sc_gather_body,
    out_type=jax.ShapeDtypeStruct((NPAD, DIM), jnp.float32),
    mesh=plsc.VectorSubcoreMesh(core_axis_name="c", subcore_axis_name="s"),
    scratch_types=[
        pltpu.VMEM((NB_W, GB), jnp.int32),
        pltpu.VMEM((NB_W, GB), jnp.int32),
        pltpu.VMEM((NB_W, GB), jnp.int32),
        pltpu.VMEM((GB, DIM), jnp.float32),
        pltpu.SemaphoreType.DMA,
    ],
    compiler_params=pltpu.CompilerParams(use_tc_tiling_on_sc=False),
)()


def kernel(colors, chars, specials, W_colors, W_chars, W_specials, W_lin, b_lin):
    table = _fuse_tables(W_colors, W_chars, W_specials, W_lin, b_lin)

    n = colors.size
    pad = NPAD - n
    idxc = jnp.pad(colors.reshape(-1).astype(jnp.int32), (0, pad))
    idxch = jnp.pad(chars.reshape(-1).astype(jnp.int32) + 16, (0, pad))
    idxs = jnp.pad(specials.reshape(-1).astype(jnp.int32) + 272, (0, pad))

    out = _sc_gather(
        idxc.reshape(NBATCH, GB),
        idxch.reshape(NBATCH, GB),
        idxs.reshape(NBATCH, GB),
        table,
    )
    return out[:n].reshape(colors.shape + (DIM,))


# speedup vs baseline: 3.3118x; 2.3720x over previous
import jax
import jax.numpy as jnp
from jax import lax
from jax.experimental import pallas as pl
from jax.experimental.pallas import tpu as pltpu
from jax.experimental.pallas import tpu_sc as plsc

DIM = 64
NPAIR = 16 * 256
NTAB = NPAIR + 256

NC = 2
NS = 16
NW = NC * NS

GB = 128
NB_W = 104
NBATCH = NW * NB_W
NPAD = NBATCH * GB
K = 4


def _fuse_tables_body(wc_ref, wch_ref, ws_ref, wl_ref, b_ref, out_ref):
    wl = wl_ref[...]
    t_c = jnp.dot(wc_ref[...], wl[0:64], preferred_element_type=jnp.float32)
    t_c = t_c + b_ref[...]
    t_ch = jnp.dot(wch_ref[...], wl[64:128], preferred_element_type=jnp.float32)
    t_s = jnp.dot(ws_ref[...], wl[128:192], preferred_element_type=jnp.float32)
    for i in range(16):
        out_ref[i * 256:(i + 1) * 256, :] = t_c[i:i + 1, :] + t_ch
    out_ref[NPAIR:NTAB, :] = t_s


def _fuse_tables(W_colors, W_chars, W_specials, W_lin, b_lin):
    return pl.pallas_call(
        _fuse_tables_body,
        out_shape=jax.ShapeDtypeStruct((NTAB, DIM), jnp.float32),
    )(W_colors, W_chars, W_specials, W_lin, b_lin.reshape(1, DIM))


def _sc_gather_body(idx1_h, idx2_h, table_h, out_h, idx1_v, idx2_v, rows_v,
                    sg0, sg1, sg2, sg3, so0, so1, so2, so3):
    semg = (sg0, sg1, sg2, sg3)
    semo = (so0, so1, so2, so3)
    wid = lax.axis_index("s") * NC + lax.axis_index("c")
    b0 = wid * NB_W
    c1 = pltpu.async_copy(idx1_h.at[pl.ds(b0, NB_W)], idx1_v, sg0)
    c2 = pltpu.async_copy(idx2_h.at[pl.ds(b0, NB_W)], idx2_v, sg1)
    c1.wait()
    c2.wait()

    def super_step(t, carry):
        for b in range(K):
            j = t * K + b
            b1 = (b - 1) % K
            b2 = (b - 2) % K

            @pl.when(j >= K)
            def _():
                pltpu.make_async_copy(
                    rows_v.at[b], out_h.at[pl.ds((b0 + j - K) * GB, GB)],
                    semo[b]).wait()

            @pl.when(j < NB_W)
            def _():
                pltpu.async_copy(table_h.at[idx1_v.at[j]], rows_v.at[b],
                                 semg[b])

            @pl.when(jnp.logical_and(j >= 1, j <= NB_W))
            def _():
                pltpu.make_async_copy(table_h.at[idx1_v.at[j - 1]],
                                      rows_v.at[b1], semg[b1]).wait()
                pltpu.async_copy(table_h.at[idx2_v.at[j - 1]], rows_v.at[b1],
                                 semg[b1], add=True)

            @pl.when(jnp.logical_and(j >= 2, j <= NB_W + 1))
            def _():
                pltpu.make_async_copy(table_h.at[idx2_v.at[j - 2]],
                                      rows_v.at[b2], semg[b2]).wait()
                pltpu.async_copy(rows_v.at[b2],
                                 out_h.at[pl.ds((b0 + j - 2) * GB, GB)],
                                 semo[b2])
        return carry

    lax.fori_loop(0, (NB_W + K) // K, super_step, 0)


_sc_gather = pl.kernel(
    _sc_gather_body,
    out_type=jax.ShapeDtypeStruct((NPAD, DIM), jnp.float32),
    mesh=plsc.VectorSubcoreMesh(core_axis_name="c", subcore_axis_name="s"),
    scratch_types=[
        pltpu.VMEM((NB_W, GB), jnp.int32),
        pltpu.VMEM((NB_W, GB), jnp.int32),
        pltpu.VMEM((K, GB, DIM), jnp.float32),
    ] + [pltpu.SemaphoreType.DMA] * (2 * K),
    compiler_params=pltpu.CompilerParams(use_tc_tiling_on_sc=False),
)


def kernel(colors, chars, specials, W_colors, W_chars, W_specials, W_lin, b_lin):
    table = _fuse_tables(W_colors, W_chars, W_specials, W_lin, b_lin)

    n = colors.size
    pad = NPAD - n
    idx1 = colors.reshape(-1).astype(jnp.int32) * 256 + chars.reshape(-1).astype(jnp.int32)
    idx2 = specials.reshape(-1).astype(jnp.int32) + NPAIR
    idx1 = jnp.pad(idx1, (0, pad))
    idx2 = jnp.pad(idx2, (0, pad))

    out = _sc_gather(
        idx1.reshape(NBATCH, GB),
        idx2.reshape(NBATCH, GB),
        table,
    )
    return out[:n].reshape(colors.shape + (DIM,))
